# baseline (speedup 1.0000x reference)
import functools

import jax
import jax.numpy as jnp
from jax import lax
from jax.experimental import pallas as pl
from jax.experimental.pallas import tpu as pltpu
from jax.experimental.pallas import tpu_sc as plsc

_DIM = 1024
_SCALE = _DIM ** (-0.5)
_NC, _NS, _L = 2, 16, 16
_NW = _NC * _NS
_SC_ROWS = 2048


def _sc_scale(emb, row0, rows):
    rows_w = rows // _NW
    ch_rows = min(32, rows_w)
    nch = rows_w // ch_rows
    nbuf = 3

    mesh = plsc.VectorSubcoreMesh(core_axis_name="c", subcore_axis_name="s")

    @functools.partial(
        pl.kernel,
        out_type=jax.ShapeDtypeStruct((rows, _DIM), jnp.float32),
        mesh=mesh,
        scratch_types=[
            pltpu.VMEM((ch_rows, _DIM), jnp.float32),
            pltpu.VMEM((ch_rows, _DIM), jnp.float32),
            pltpu.VMEM((ch_rows, _DIM), jnp.float32),
            pltpu.SemaphoreType.DMA,
            pltpu.SemaphoreType.DMA,
            pltpu.SemaphoreType.DMA,
            pltpu.SemaphoreType.DMA,
            pltpu.SemaphoreType.DMA,
            pltpu.SemaphoreType.DMA,
        ],
    )
    def k(emb_hbm, out_hbm, buf0, buf1, buf2, si0, si1, si2, so0, so1, so2):
        wid = lax.axis_index("s") * _NC + lax.axis_index("c")
        base = wid * rows_w
        bufs = (buf0, buf1, buf2)
        sin = (si0, si1, si2)
        sout = (so0, so1, so2)

        def in_copy(ch):
            b = ch % nbuf
            return pltpu.async_copy(
                emb_hbm.at[pl.ds(row0 + base + ch * ch_rows, ch_rows)],
                bufs[b], sin[b])

        def out_copy(ch):
            b = ch % nbuf
            return pltpu.async_copy(
                bufs[b], out_hbm.at[pl.ds(base + ch * ch_rows, ch_rows)],
                sout[b])

        in_d = {ch: in_copy(ch) for ch in range(min(nbuf, nch))}
        out_d = {}
        for ch in range(nch):
            if ch >= nbuf - 1 and ch + 1 < nch:
                out_d[ch - (nbuf - 1)].wait()
                in_d[ch + 1] = in_copy(ch + 1)
            in_d[ch].wait()
            buf = bufs[ch % nbuf]

            @plsc.parallel_loop(0, ch_rows)
            def _body(r):
                for cc in range(0, _DIM, _L):
                    buf[r, pl.ds(cc, _L)] = buf[r, pl.ds(cc, _L)] * _SCALE

            out_d[ch] = out_copy(ch)

        for ch in range(max(0, nch - nbuf), nch):
            out_d[ch].wait()

    return k(emb)


def _tc_body(e_ref, o_ref):
    o_ref[...] = e_ref[...] * _SCALE


def _tc_scale(emb, rows):
    block = 1024
    return pl.pallas_call(
        _tc_body,
        grid=(rows // block,),
        in_specs=[pl.BlockSpec((block, _DIM), lambda i: (i, 0))],
        out_specs=pl.BlockSpec((block, _DIM), lambda i: (i, 0)),
        out_shape=jax.ShapeDtypeStruct((rows, _DIM), emb.dtype),
    )(emb)


def kernel(x, emb):
    seq_len = x.shape[1]
    emb = emb[:seq_len]
    tc_rows = seq_len - _SC_ROWS
    bottom = _sc_scale(emb, tc_rows, _SC_ROWS)
    top = _tc_scale(emb, tc_rows)
    return jnp.concatenate([top, bottom], axis=0)


# speedup vs baseline: 1.7994x; 1.7994x over previous
import jax
import jax.numpy as jnp
from jax.experimental import pallas as pl

_DIM = 1024
_SCALE = _DIM ** (-0.5)


def _scale_body(e_ref, o_ref):
    o_ref[...] = e_ref[...] * _SCALE


def kernel(x, emb):
    seq_len = x.shape[1]
    rows_per_block = 256
    grid = (seq_len // rows_per_block,)
    return pl.pallas_call(
        _scale_body,
        grid=grid,
        in_specs=[pl.BlockSpec((rows_per_block, _DIM), lambda i: (i, 0))],
        out_specs=pl.BlockSpec((rows_per_block, _DIM), lambda i: (i, 0)),
        out_shape=jax.ShapeDtypeStruct((seq_len, _DIM), emb.dtype),
    )(emb[:seq_len])


# speedup vs baseline: 2.8721x; 1.5962x over previous
import jax
import jax.numpy as jnp
from jax.experimental import pallas as pl

_DIM = 1024
_SCALE = _DIM ** (-0.5)


def _scale_body(e_ref, o_ref):
    o_ref[...] = e_ref[...] * _SCALE


def kernel(x, emb):
    seq_len = x.shape[1]
    rows_per_block = 2048
    grid = (seq_len // rows_per_block,)
    return pl.pallas_call(
        _scale_body,
        grid=grid,
        in_specs=[pl.BlockSpec((rows_per_block, _DIM), lambda i: (i, 0))],
        out_specs=pl.BlockSpec((rows_per_block, _DIM), lambda i: (i, 0)),
        out_shape=jax.ShapeDtypeStruct((seq_len, _DIM), emb.dtype),
    )(emb[:seq_len])
